# baseline (speedup 1.0000x reference)
import jax
import jax.numpy as jnp
from jax.experimental import pallas as pl
from jax.experimental.pallas import tpu as pltpu
from jax.experimental.pallas import tpu_sc as plsc


def _dot(x, y, dims):
    return jax.lax.dot_general(x, y, (dims, ((), ())),
                               preferred_element_type=jnp.float32)


def _bdot(x, y, dims):
    return _dot(x.astype(jnp.bfloat16), y.astype(jnp.bfloat16), dims)


def _split3(x):
    a = x.astype(jnp.bfloat16)
    r1 = x - a.astype(jnp.float32)
    b = r1.astype(jnp.bfloat16)
    c = (r1 - b.astype(jnp.float32)).astype(jnp.bfloat16)
    return a, b, c


def _gather3(oh, cba, cbb, cbc):
    dims = ((1,), (0,))
    return (_dot(oh, cba, dims) + _dot(oh, cbb, dims)) + _dot(oh, cbc, dims)


def _rowsq_body(cb_ref, out_ref):
    cb = cb_ref[...]
    a, b, c = _split3(cb * cb)
    ones = jnp.ones((1, cb.shape[1]), jnp.bfloat16)
    dims = ((1,), (1,))
    out_ref[...] = (_dot(ones, a, dims) + _dot(ones, b, dims)) + _dot(
        ones, c, dims)


def _rowsq(cb, tk=4096):
    k, d = cb.shape
    tk = min(tk, k)
    return pl.pallas_call(
        _rowsq_body,
        grid=(k // tk,),
        in_specs=[pl.BlockSpec((tk, d), lambda i: (i, 0))],
        out_specs=pl.BlockSpec((1, tk), lambda i: (0, i)),
        out_shape=jax.ShapeDtypeStruct((1, k), jnp.float32),
    )(cb)


def _split3_body(x_ref, a_ref, b_ref, c_ref):
    a, b, c = _split3(x_ref[...])
    a_ref[...] = a
    b_ref[...] = b
    c_ref[...] = c


def _split3_k(x):
    sh = jax.ShapeDtypeStruct(x.shape, jnp.bfloat16)
    return pl.pallas_call(_split3_body, out_shape=[sh, sh, sh])(x)


def _vq2_body(z_ref, cbsn2_ref, cbsq_ref, iot_ref, si_ref, di_ref):
    k = cbsn2_ref.shape[0] // 2
    z = z_ref[...]
    zsq = jnp.sum(z * z, axis=1, keepdims=True)
    zn2 = _dot(z.astype(jnp.bfloat16), cbsn2_ref[...], (((1,), (1,))))
    dist = (zsq + zn2) + cbsq_ref[...]
    iot = iot_ref[...]
    for half, ref in ((dist[:, :k], si_ref), (dist[:, k:], di_ref)):
        m = jnp.min(half, axis=1, keepdims=True)
        ref[0, 0, :] = jnp.min(
            jnp.where(half == m, iot, jnp.float32(k)), axis=1).astype(jnp.int32)


def _vq_argmin2(z, cbs16, cbsq2, iota_f, tm):
    n, d = z.shape
    k2 = cbs16.shape[0]
    nt = n // tm
    ispec = pl.BlockSpec((1, 1, tm), lambda i: (i, 0, 0))
    ishape = jax.ShapeDtypeStruct((nt, 1, tm), jnp.int32)
    si3, di3 = pl.pallas_call(
        _vq2_body,
        grid=(nt,),
        in_specs=[
            pl.BlockSpec((tm, d), lambda i: (i, 0)),
            pl.BlockSpec((k2, d), lambda i: (0, 0)),
            pl.BlockSpec((1, k2), lambda i: (0, 0)),
            pl.BlockSpec((1, k2 // 2), lambda i: (0, 0)),
        ],
        out_specs=[ispec, ispec],
        out_shape=[ishape, ishape],
    )(z, cbs16, cbsq2, iota_f)
    return si3.reshape(n), di3.reshape(n)


_GW = 128


def _sc_gather2(tab_a, idx_a, tab_b, idx_b):
    n = idx_a.shape[0]
    d = tab_a.shape[1]
    ia = idx_a.reshape(1, n)
    ib = idx_b.reshape(1, n)
    mesh = plsc.VectorSubcoreMesh(core_axis_name="core",
                                  subcore_axis_name="subcore")
    out = jax.ShapeDtypeStruct((n, d), tab_a.dtype)

    @pl.kernel(out_type=(out, out), mesh=mesh)
    def run(ta_hbm, ia_hbm, tb_hbm, ib_hbm, oa_hbm, ob_hbm):
        ispec = pl.BlockSpec((1, _GW), lambda i: (0, i))
        ospec = pl.BlockSpec((_GW, d), lambda i: (i, 0))
        for tab, ih, oh in ((ta_hbm, ia_hbm, oa_hbm), (tb_hbm, ib_hbm, ob_hbm)):
            def body(i_vmem, o_vmem, tab=tab):
                pltpu.sync_copy(tab.at[i_vmem.at[0]], o_vmem)

            pltpu.emit_pipeline(
                body,
                grid=(n // _GW,),
                in_specs=[ispec],
                out_specs=[ospec],
                core_axis_name=("core", "subcore"),
                dimension_semantics=(pltpu.PARALLEL,),
            )(ih, oh)

    return run(tab_a, ia, tab_b, ib)


def _mm1_body(z_ref, qs_ref, qd_ref, w1_ref, b1_ref, h_ref, part_ref):
    z = z_ref[...]
    qs = qs_ref[...]
    qd = qd_ref[...]
    comb = jnp.concatenate([z + (qs - z), z + (qd - z)], axis=1)
    h_ref[...] = _bdot(comb, w1_ref[...], (((1,), (1,)))) + b1_ref[...]
    part_ref[...] = jnp.concatenate(
        [jnp.sum((z - qs) ** 2, axis=0, keepdims=True),
         jnp.sum((z - qd) ** 2, axis=0, keepdims=True)], axis=0)[None]


def _mm1(z, q_sem, q_det, w1, b1, tm):
    n, d = z.shape
    nt = n // tm
    row = lambda i: (i, 0)
    h, parts = pl.pallas_call(
        _mm1_body,
        grid=(nt,),
        in_specs=[
            pl.BlockSpec((tm, d), row),
            pl.BlockSpec((tm, d), row),
            pl.BlockSpec((tm, d), row),
            pl.BlockSpec((d, 2 * d), lambda i: (0, 0)),
            pl.BlockSpec((1, d), lambda i: (0, 0)),
        ],
        out_specs=[
            pl.BlockSpec((tm, d), row),
            pl.BlockSpec((1, 2, d), lambda i: (i, 0, 0)),
        ],
        out_shape=[
            jax.ShapeDtypeStruct((n, d), jnp.float32),
            jax.ShapeDtypeStruct((nt, 2, d), jnp.float32),
        ],
    )(z, q_sem, q_det, w1, b1.reshape(1, d))
    return h, parts


def _ln_gelu_mm2_body(h_ref, m_ref, v_ref, g_ref, b_ref, w2_ref, b2_ref,
                      out_ref):
    h = h_ref[...]
    ln = ((h - m_ref[...]) / jnp.sqrt(v_ref[...] + 1e-5) * g_ref[...]
          + b_ref[...])
    act = jax.nn.gelu(ln)
    out_ref[...] = _bdot(act, w2_ref[...], (((1,), (1,)))) + b2_ref[...]


def _ln_gelu_mm2(h, m, v, g, b, w2, b2, tm):
    n, d = h.shape
    nt = n // tm
    row = lambda i: (i, 0)
    const = lambda i: (0, 0)
    return pl.pallas_call(
        _ln_gelu_mm2_body,
        grid=(nt,),
        in_specs=[
            pl.BlockSpec((tm, d), row),
            pl.BlockSpec((tm, 1), row),
            pl.BlockSpec((tm, 1), row),
            pl.BlockSpec((1, d), const),
            pl.BlockSpec((1, d), const),
            pl.BlockSpec((d, d), const),
            pl.BlockSpec((1, d), const),
        ],
        out_specs=pl.BlockSpec((tm, d), row),
        out_shape=jax.ShapeDtypeStruct((n, d), jnp.float32),
    )(h, m, v, g.reshape(1, d), b.reshape(1, d), w2, b2.reshape(1, d))


def _rvq_body(h_ref, m_ref, v_ref, g_ref, b_ref, ra_ref, rb_ref, rc_ref,
              rsq_ref, iot_ref, wo_ref, bo_ref, out_ref, i0_ref, i1_ref,
              i2_ref, i3_ref, part_ref):
    h = h_ref[...]
    refined = ((h - m_ref[...]) / jnp.sqrt(v_ref[...] + 1e-5) * g_ref[...]
               + b_ref[...])
    idx_refs = (i0_ref, i1_ref, i2_ref, i3_ref)
    krvq = ra_ref.shape[1]
    r = refined
    quant = jnp.zeros_like(refined)
    parts = []
    for s in range(ra_ref.shape[0]):
        ca, cb, cc = ra_ref[s], rb_ref[s], rc_ref[s]
        rsq = jnp.sum(r * r, axis=1, keepdims=True)
        zn = _dot(r.astype(jnp.bfloat16), ca, (((1,), (1,))))
        dist = (rsq - 2.0 * zn) + rsq_ref[s]
        mn = jnp.min(dist, axis=1, keepdims=True)
        iot = iot_ref[...]
        idxf = jnp.min(jnp.where(dist == mn, iot, jnp.float32(krvq)), axis=1)
        idx_refs[s][0, 0, :] = idxf.astype(jnp.int32)
        oh = (iot == idxf[:, None]).astype(jnp.bfloat16)
        q = _gather3(oh, ca, cb, cc)
        parts.append(jnp.sum((r - q) ** 2, axis=0, keepdims=True))
        q_st = r + (q - r)
        r = r - q_st
        quant = quant + q_st
    out_ref[...] = _bdot(quant, wo_ref[...], (((1,), (1,)))) + bo_ref[...]
    part_ref[...] = jnp.concatenate(parts, axis=0)[None]


def _rvq_out(h, m, v, g, b, ra, rb, rc, rvqsq, iota_f, wo, bo, tm):
    n, d = h.shape
    nt = n // tm
    nq, krvq, _ = ra.shape
    full3 = lambda i: (0, 0, 0)
    const = lambda i: (0, 0)
    row = lambda i: (i, 0)
    ispec = pl.BlockSpec((1, 1, tm), lambda i: (i, 0, 0))
    rspec = pl.BlockSpec((nq, krvq, d), full3)
    out, i0, i1, i2, i3, parts = pl.pallas_call(
        _rvq_body,
        grid=(nt,),
        in_specs=[
            pl.BlockSpec((tm, d), row),
            pl.BlockSpec((tm, 1), row),
            pl.BlockSpec((tm, 1), row),
            pl.BlockSpec((1, d), const),
            pl.BlockSpec((1, d), const),
            rspec, rspec, rspec,
            pl.BlockSpec((nq, 1, krvq), full3),
            pl.BlockSpec((1, krvq), const),
            pl.BlockSpec((d, d), const),
            pl.BlockSpec((1, d), const),
        ],
        out_specs=[
            pl.BlockSpec((tm, d), row),
            ispec, ispec, ispec, ispec,
            pl.BlockSpec((1, 4, d), lambda i: (i, 0, 0)),
        ],
        out_shape=[
            jax.ShapeDtypeStruct((n, d), jnp.float32),
            jax.ShapeDtypeStruct((nt, 1, tm), jnp.int32),
            jax.ShapeDtypeStruct((nt, 1, tm), jnp.int32),
            jax.ShapeDtypeStruct((nt, 1, tm), jnp.int32),
            jax.ShapeDtypeStruct((nt, 1, tm), jnp.int32),
            jax.ShapeDtypeStruct((nt, 4, d), jnp.float32),
        ],
    )(h, m, v, g.reshape(1, d), b.reshape(1, d), ra, rb, rc, rvqsq, iota_f,
      wo, bo.reshape(1, d))
    res_idx = jnp.stack([i0.reshape(n), i1.reshape(n), i2.reshape(n),
                         i3.reshape(n)], axis=-1)
    return out, res_idx, parts


def kernel(z, sem_codebook, det_codebook, w1, b1, ln1_g, ln1_b, w2, b2,
           ln2_g, ln2_b, rvq_codebooks, w_out, b_out):
    n, d = z.shape
    nq, krvq, _ = rvq_codebooks.shape
    k = sem_codebook.shape[0]
    allcb = jnp.concatenate(
        [sem_codebook, det_codebook, rvq_codebooks.reshape(nq * krvq, d)],
        axis=0)
    allsq = _rowsq(allcb)
    cbsq2 = allsq[:, :2 * k]
    rvq_sq = allsq[:, 2 * k:].reshape(nq, 1, krvq)
    ra, rb, rc = _split3_k(rvq_codebooks.reshape(nq * krvq, d))
    ra = ra.reshape(nq, krvq, d)
    rb = rb.reshape(nq, krvq, d)
    rc = rc.reshape(nq, krvq, d)

    iota_f = jnp.arange(k, dtype=jnp.float32).reshape(1, k)
    cbsn2 = allcb[:2 * k].astype(jnp.bfloat16) * jnp.bfloat16(-2.0)
    sem_idx, det_idx = _vq_argmin2(z, cbsn2, cbsq2, iota_f, 256)
    q_sem, q_det = _sc_gather2(sem_codebook, sem_idx, det_codebook, det_idx)

    h1, commit_parts = _mm1(z, q_sem, q_det, w1, b1, 1024)
    m1 = jnp.mean(h1, axis=-1, keepdims=True)
    v1 = jnp.var(h1, axis=-1, keepdims=True)
    h2 = _ln_gelu_mm2(h1, m1, v1, ln1_g, ln1_b, w2, b2, 1024)
    m2 = jnp.mean(h2, axis=-1, keepdims=True)
    v2 = jnp.var(h2, axis=-1, keepdims=True)
    out, res_idx, rvq_parts = _rvq_out(h2, m2, v2, ln2_g, ln2_b, ra, rb, rc,
                                       rvq_sq, iota_f[:, :krvq], w_out,
                                       b_out, 2048)

    csums = jnp.sum(commit_parts, axis=(0, 2))
    rsums = jnp.sum(rvq_parts, axis=(0, 2))
    denom = jnp.float32(n * d)
    vq_loss = (0.25 * csums[0] / denom + 0.375 * csums[1] / denom
               + (rsums[0] + rsums[1] + rsums[2] + rsums[3]) / denom)
    return out, sem_idx, det_idx, res_idx, vq_loss.astype(jnp.float32)


# speedup vs baseline: 1.0256x; 1.0256x over previous
import jax
import jax.numpy as jnp
from jax.experimental import pallas as pl
from jax.experimental.pallas import tpu as pltpu
from jax.experimental.pallas import tpu_sc as plsc


def _dot(x, y, dims):
    return jax.lax.dot_general(x, y, (dims, ((), ())),
                               preferred_element_type=jnp.float32)


def _bdot(x, y, dims):
    return _dot(x.astype(jnp.bfloat16), y.astype(jnp.bfloat16), dims)


def _split3(x):
    a = x.astype(jnp.bfloat16)
    r1 = x - a.astype(jnp.float32)
    b = r1.astype(jnp.bfloat16)
    c = (r1 - b.astype(jnp.float32)).astype(jnp.bfloat16)
    return a, b, c


def _gather3(oh, cba, cbb, cbc):
    dims = ((1,), (0,))
    return (_dot(oh, cba, dims) + _dot(oh, cbb, dims)) + _dot(oh, cbc, dims)


def _rowsq_body(cb_ref, out_ref):
    cb = cb_ref[...]
    a, b, c = _split3(cb * cb)
    ones = jnp.ones((1, cb.shape[1]), jnp.bfloat16)
    dims = ((1,), (1,))
    out_ref[...] = (_dot(ones, a, dims) + _dot(ones, b, dims)) + _dot(
        ones, c, dims)


def _rowsq(cb, tk=4096):
    k, d = cb.shape
    tk = min(tk, k)
    return pl.pallas_call(
        _rowsq_body,
        grid=(k // tk,),
        in_specs=[pl.BlockSpec((tk, d), lambda i: (i, 0))],
        out_specs=pl.BlockSpec((1, tk), lambda i: (0, i)),
        out_shape=jax.ShapeDtypeStruct((1, k), jnp.float32),
    )(cb)


def _split3_body(x_ref, a_ref, b_ref, c_ref):
    a, b, c = _split3(x_ref[...])
    a_ref[...] = a
    b_ref[...] = b
    c_ref[...] = c


def _split3_k(x):
    sh = jax.ShapeDtypeStruct(x.shape, jnp.bfloat16)
    return pl.pallas_call(_split3_body, out_shape=[sh, sh, sh])(x)


def _vq2_body(z_ref, cbsn2_ref, cbsq_ref, iot_ref, si_ref, di_ref):
    k = cbsn2_ref.shape[0] // 2
    z = z_ref[...]
    zsq = jnp.sum(z * z, axis=1, keepdims=True)
    zn2 = _dot(z.astype(jnp.bfloat16), cbsn2_ref[...], (((1,), (1,))))
    dist = (zsq + zn2) + cbsq_ref[...]
    iot = iot_ref[...]
    for half, ref in ((dist[:, :k], si_ref), (dist[:, k:], di_ref)):
        m = jnp.min(half, axis=1, keepdims=True)
        ref[0, 0, :] = jnp.min(
            jnp.where(half == m, iot, jnp.float32(k)), axis=1).astype(jnp.int32)


def _vq_argmin2(z, cbs16, cbsq2, iota_f, tm):
    n, d = z.shape
    k2 = cbs16.shape[0]
    nt = n // tm
    ispec = pl.BlockSpec((1, 1, tm), lambda i: (i, 0, 0))
    ishape = jax.ShapeDtypeStruct((nt, 1, tm), jnp.int32)
    si3, di3 = pl.pallas_call(
        _vq2_body,
        grid=(nt,),
        in_specs=[
            pl.BlockSpec((tm, d), lambda i: (i, 0)),
            pl.BlockSpec((k2, d), lambda i: (0, 0)),
            pl.BlockSpec((1, k2), lambda i: (0, 0)),
            pl.BlockSpec((1, k2 // 2), lambda i: (0, 0)),
        ],
        out_specs=[ispec, ispec],
        out_shape=[ishape, ishape],
    )(z, cbs16, cbsq2, iota_f)
    return si3.reshape(n), di3.reshape(n)


_GW = 128


def _sc_gather(table, idx):
    n = idx.shape[0]
    d = table.shape[1]
    idx2 = idx.reshape(1, n)
    mesh = plsc.VectorSubcoreMesh(core_axis_name="core",
                                  subcore_axis_name="subcore")

    @pl.kernel(out_type=jax.ShapeDtypeStruct((n, d), table.dtype), mesh=mesh)
    def run(tab_hbm, i_hbm, o_hbm):
        def body(i_vmem, o_vmem):
            pltpu.sync_copy(tab_hbm.at[i_vmem.at[0]], o_vmem)

        pltpu.emit_pipeline(
            body,
            grid=(n // _GW,),
            in_specs=[pl.BlockSpec((1, _GW), lambda i: (0, i))],
            out_specs=[pl.BlockSpec((_GW, d), lambda i: (i, 0))],
            core_axis_name=("core", "subcore"),
            dimension_semantics=(pltpu.PARALLEL,),
        )(i_hbm, o_hbm)

    return run(table, idx2)


def _mm1_body(z_ref, qs_ref, qd_ref, w1_ref, b1_ref, h_ref, part_ref):
    z = z_ref[...]
    qs = qs_ref[...]
    qd = qd_ref[...]
    comb = jnp.concatenate([z + (qs - z), z + (qd - z)], axis=1)
    h_ref[...] = _bdot(comb, w1_ref[...], (((1,), (1,)))) + b1_ref[...]
    part_ref[...] = jnp.concatenate(
        [jnp.sum((z - qs) ** 2, axis=0, keepdims=True),
         jnp.sum((z - qd) ** 2, axis=0, keepdims=True)], axis=0)[None]


def _mm1(z, q_sem, q_det, w1, b1, tm):
    n, d = z.shape
    nt = n // tm
    row = lambda i: (i, 0)
    h, parts = pl.pallas_call(
        _mm1_body,
        grid=(nt,),
        in_specs=[
            pl.BlockSpec((tm, d), row),
            pl.BlockSpec((tm, d), row),
            pl.BlockSpec((tm, d), row),
            pl.BlockSpec((d, 2 * d), lambda i: (0, 0)),
            pl.BlockSpec((1, d), lambda i: (0, 0)),
        ],
        out_specs=[
            pl.BlockSpec((tm, d), row),
            pl.BlockSpec((1, 2, d), lambda i: (i, 0, 0)),
        ],
        out_shape=[
            jax.ShapeDtypeStruct((n, d), jnp.float32),
            jax.ShapeDtypeStruct((nt, 2, d), jnp.float32),
        ],
    )(z, q_sem, q_det, w1, b1.reshape(1, d))
    return h, parts


def _ln_gelu_mm2_body(h_ref, m_ref, v_ref, g_ref, b_ref, w2_ref, b2_ref,
                      out_ref):
    h = h_ref[...]
    ln = ((h - m_ref[...]) / jnp.sqrt(v_ref[...] + 1e-5) * g_ref[...]
          + b_ref[...])
    act = jax.nn.gelu(ln)
    out_ref[...] = _bdot(act, w2_ref[...], (((1,), (1,)))) + b2_ref[...]


def _ln_gelu_mm2(h, m, v, g, b, w2, b2, tm):
    n, d = h.shape
    nt = n // tm
    row = lambda i: (i, 0)
    const = lambda i: (0, 0)
    return pl.pallas_call(
        _ln_gelu_mm2_body,
        grid=(nt,),
        in_specs=[
            pl.BlockSpec((tm, d), row),
            pl.BlockSpec((tm, 1), row),
            pl.BlockSpec((tm, 1), row),
            pl.BlockSpec((1, d), const),
            pl.BlockSpec((1, d), const),
            pl.BlockSpec((d, d), const),
            pl.BlockSpec((1, d), const),
        ],
        out_specs=pl.BlockSpec((tm, d), row),
        out_shape=jax.ShapeDtypeStruct((n, d), jnp.float32),
    )(h, m, v, g.reshape(1, d), b.reshape(1, d), w2, b2.reshape(1, d))


def _rvq_body(h_ref, m_ref, v_ref, g_ref, b_ref, ra_ref, rb_ref, rc_ref,
              rsq_ref, iot_ref, wo_ref, bo_ref, out_ref, i0_ref, i1_ref,
              i2_ref, i3_ref, part_ref):
    h = h_ref[...]
    refined = ((h - m_ref[...]) / jnp.sqrt(v_ref[...] + 1e-5) * g_ref[...]
               + b_ref[...])
    idx_refs = (i0_ref, i1_ref, i2_ref, i3_ref)
    krvq = ra_ref.shape[1]
    r = refined
    quant = jnp.zeros_like(refined)
    parts = []
    for s in range(ra_ref.shape[0]):
        ca, cb, cc = ra_ref[s], rb_ref[s], rc_ref[s]
        rsq = jnp.sum(r * r, axis=1, keepdims=True)
        zn = _dot(r.astype(jnp.bfloat16), ca, (((1,), (1,))))
        dist = (rsq - 2.0 * zn) + rsq_ref[s]
        mn = jnp.min(dist, axis=1, keepdims=True)
        iot = iot_ref[...]
        idxf = jnp.min(jnp.where(dist == mn, iot, jnp.float32(krvq)), axis=1)
        idx_refs[s][0, 0, :] = idxf.astype(jnp.int32)
        oh = (iot == idxf[:, None]).astype(jnp.bfloat16)
        q = _gather3(oh, ca, cb, cc)
        parts.append(jnp.sum((r - q) ** 2, axis=0, keepdims=True))
        q_st = r + (q - r)
        r = r - q_st
        quant = quant + q_st
    out_ref[...] = _bdot(quant, wo_ref[...], (((1,), (1,)))) + bo_ref[...]
    part_ref[...] = jnp.concatenate(parts, axis=0)[None]


def _rvq_out(h, m, v, g, b, ra, rb, rc, rvqsq, iota_f, wo, bo, tm):
    n, d = h.shape
    nt = n // tm
    nq, krvq, _ = ra.shape
    full3 = lambda i: (0, 0, 0)
    const = lambda i: (0, 0)
    row = lambda i: (i, 0)
    ispec = pl.BlockSpec((1, 1, tm), lambda i: (i, 0, 0))
    rspec = pl.BlockSpec((nq, krvq, d), full3)
    out, i0, i1, i2, i3, parts = pl.pallas_call(
        _rvq_body,
        grid=(nt,),
        in_specs=[
            pl.BlockSpec((tm, d), row),
            pl.BlockSpec((tm, 1), row),
            pl.BlockSpec((tm, 1), row),
            pl.BlockSpec((1, d), const),
            pl.BlockSpec((1, d), const),
            rspec, rspec, rspec,
            pl.BlockSpec((nq, 1, krvq), full3),
            pl.BlockSpec((1, krvq), const),
            pl.BlockSpec((d, d), const),
            pl.BlockSpec((1, d), const),
        ],
        out_specs=[
            pl.BlockSpec((tm, d), row),
            ispec, ispec, ispec, ispec,
            pl.BlockSpec((1, 4, d), lambda i: (i, 0, 0)),
        ],
        out_shape=[
            jax.ShapeDtypeStruct((n, d), jnp.float32),
            jax.ShapeDtypeStruct((nt, 1, tm), jnp.int32),
            jax.ShapeDtypeStruct((nt, 1, tm), jnp.int32),
            jax.ShapeDtypeStruct((nt, 1, tm), jnp.int32),
            jax.ShapeDtypeStruct((nt, 1, tm), jnp.int32),
            jax.ShapeDtypeStruct((nt, 4, d), jnp.float32),
        ],
    )(h, m, v, g.reshape(1, d), b.reshape(1, d), ra, rb, rc, rvqsq, iota_f,
      wo, bo.reshape(1, d))
    res_idx = jnp.stack([i0.reshape(n), i1.reshape(n), i2.reshape(n),
                         i3.reshape(n)], axis=-1)
    return out, res_idx, parts


def kernel(z, sem_codebook, det_codebook, w1, b1, ln1_g, ln1_b, w2, b2,
           ln2_g, ln2_b, rvq_codebooks, w_out, b_out):
    n, d = z.shape
    nq, krvq, _ = rvq_codebooks.shape
    k = sem_codebook.shape[0]
    allcb = jnp.concatenate(
        [sem_codebook, det_codebook, rvq_codebooks.reshape(nq * krvq, d)],
        axis=0)
    allsq = _rowsq(allcb)
    cbsq2 = allsq[:, :2 * k]
    rvq_sq = allsq[:, 2 * k:].reshape(nq, 1, krvq)
    ra, rb, rc = _split3_k(rvq_codebooks.reshape(nq * krvq, d))
    ra = ra.reshape(nq, krvq, d)
    rb = rb.reshape(nq, krvq, d)
    rc = rc.reshape(nq, krvq, d)

    iota_f = jnp.arange(k, dtype=jnp.float32).reshape(1, k)
    cbsn2 = allcb[:2 * k].astype(jnp.bfloat16) * jnp.bfloat16(-2.0)
    sem_idx, det_idx = _vq_argmin2(z, cbsn2, cbsq2, iota_f, 256)
    q_sem = _sc_gather(sem_codebook, sem_idx)
    q_det = _sc_gather(det_codebook, det_idx)

    h1, commit_parts = _mm1(z, q_sem, q_det, w1, b1, 1024)
    m1 = jnp.mean(h1, axis=-1, keepdims=True)
    v1 = jnp.var(h1, axis=-1, keepdims=True)
    h2 = _ln_gelu_mm2(h1, m1, v1, ln1_g, ln1_b, w2, b2, 1024)
    m2 = jnp.mean(h2, axis=-1, keepdims=True)
    v2 = jnp.var(h2, axis=-1, keepdims=True)
    out, res_idx, rvq_parts = _rvq_out(h2, m2, v2, ln2_g, ln2_b, ra, rb, rc,
                                       rvq_sq, iota_f[:, :krvq], w_out,
                                       b_out, 1024)

    csums = jnp.sum(commit_parts, axis=(0, 2))
    rsums = jnp.sum(rvq_parts, axis=(0, 2))
    denom = jnp.float32(n * d)
    vq_loss = (0.25 * csums[0] / denom + 0.375 * csums[1] / denom
               + (rsums[0] + rsums[1] + rsums[2] + rsums[3]) / denom)
    return out, sem_idx, det_idx, res_idx, vq_loss.astype(jnp.float32)
